# baseline (speedup 1.0000x reference)
import math

import jax
import jax.numpy as jnp
from jax.experimental import pallas as pl


_D_MODEL = 1024
_HALF = _D_MODEL // 2
_SCALE = -math.log(10000.0) / (_HALF - 1)


def _pe_add_body(x_ref, o_ref, *, blk):
    i = pl.program_id(0)
    pos = (
        jax.lax.broadcasted_iota(jnp.int32, (blk, _HALF), 0) + i * blk
    ).astype(jnp.float32)
    j = jax.lax.broadcasted_iota(jnp.int32, (1, _HALF), 1).astype(jnp.float32)
    ang = pos * jnp.exp(j * _SCALE)
    o_ref[:, :, :_HALF] = x_ref[:, :, :_HALF] + jnp.sin(ang)[None]
    o_ref[:, :, _HALF:] = x_ref[:, :, _HALF:] + jnp.cos(ang)[None]


def kernel(inputs):
    batch, seq_len, d_model = inputs.shape
    assert d_model == _D_MODEL
    blk = 512
    grid = (seq_len // blk,)
    body = lambda x_ref, o_ref: _pe_add_body(x_ref, o_ref, blk=blk)
    return pl.pallas_call(
        body,
        grid=grid,
        in_specs=[
            pl.BlockSpec((batch, blk, d_model), lambda i: (0, i, 0)),
        ],
        out_specs=pl.BlockSpec((batch, blk, d_model), lambda i: (0, i, 0)),
        out_shape=jax.ShapeDtypeStruct(inputs.shape, inputs.dtype),
    )(inputs)


# speedup vs baseline: 1.1296x; 1.1296x over previous
import math

import jax
import jax.numpy as jnp
from jax.experimental import pallas as pl


_D_MODEL = 1024
_HALF = _D_MODEL // 2
_SCALE = -math.log(10000.0) / (_HALF - 1)


_T = 32


def _pe_add_body(x_ref, o_ref, *, blk):
    i = pl.program_id(0)
    q_grp = blk // _T
    j = jax.lax.broadcasted_iota(jnp.int32, (1, 1, _HALF), 2).astype(jnp.float32)
    div = jnp.exp(j * _SCALE)
    alpha = (
        jax.lax.broadcasted_iota(jnp.int32, (q_grp, 1, _HALF), 0) * _T + i * blk
    ).astype(jnp.float32) * div
    beta = (
        jax.lax.broadcasted_iota(jnp.int32, (1, _T, _HALF), 1)
    ).astype(jnp.float32) * div
    sa, ca = jnp.sin(alpha), jnp.cos(alpha)
    sb, cb = jnp.sin(beta), jnp.cos(beta)
    pe_sin = (sa * cb + ca * sb).reshape(blk, _HALF)
    pe_cos = (ca * cb - sa * sb).reshape(blk, _HALF)
    o_ref[:, :, :_HALF] = x_ref[:, :, :_HALF] + pe_sin[None]
    o_ref[:, :, _HALF:] = x_ref[:, :, _HALF:] + pe_cos[None]


def kernel(inputs):
    batch, seq_len, d_model = inputs.shape
    assert d_model == _D_MODEL
    blk = 512
    grid = (seq_len // blk,)
    body = lambda x_ref, o_ref: _pe_add_body(x_ref, o_ref, blk=blk)
    return pl.pallas_call(
        body,
        grid=grid,
        in_specs=[
            pl.BlockSpec((batch, blk, d_model), lambda i: (0, i, 0)),
        ],
        out_specs=pl.BlockSpec((batch, blk, d_model), lambda i: (0, i, 0)),
        out_shape=jax.ShapeDtypeStruct(inputs.shape, inputs.dtype),
    )(inputs)
